# baseline (speedup 1.0000x reference)
import functools

import jax
import jax.numpy as jnp
from jax.experimental import pallas as pl
from jax.experimental.pallas import tpu as pltpu


def _gcn_body(x_ref, adj_ref, W1_ref, b1_ref, W2_ref, b2_ref, out_ref,
              s1_ref, h_ref, t_ref, *, BM):
    p = pl.program_id(0)
    m = pl.program_id(1)

    @pl.when((p == 0) & (m == 0))
    def _():
        s1_ref[...] = jnp.dot(x_ref[...], W1_ref[...],
                              preferred_element_type=jnp.float32)

    @pl.when(p == 0)
    def _():
        acc = jnp.dot(adj_ref[...], s1_ref[...],
                      preferred_element_type=jnp.float32)
        h_ref[pl.ds(m * BM, BM), :] = jnp.maximum(acc + b1_ref[...], 0.0)
        out_ref[...] = jnp.zeros_like(out_ref)

    @pl.when((p == 1) & (m == 0))
    def _():
        t_ref[...] = jnp.dot(h_ref[...], W2_ref[...],
                             preferred_element_type=jnp.float32)

    @pl.when(p == 1)
    def _():
        o = jnp.dot(adj_ref[...], t_ref[...],
                    preferred_element_type=jnp.float32) + b2_ref[...]
        o = o - jnp.max(o, axis=1, keepdims=True)
        out_ref[...] = o - jnp.log(jnp.sum(jnp.exp(o), axis=1, keepdims=True))


def _pick_bm(n):
    for bm in (400, 200, 80, 40, 8):
        if n % bm == 0:
            return bm
    return n


@jax.jit
def kernel(x, adj, W1, b1, W2, b2):
    N, F = x.shape
    H = W1.shape[1]
    C = W2.shape[1]
    BM = _pick_bm(N)
    grid = (2, N // BM)

    out = pl.pallas_call(
        functools.partial(_gcn_body, BM=BM),
        grid=grid,
        in_specs=[
            pl.BlockSpec((N, F), lambda p, m: (0, 0)),
            pl.BlockSpec((BM, N), lambda p, m: (m, 0)),
            pl.BlockSpec((F, H), lambda p, m: (0, 0)),
            pl.BlockSpec((1, H), lambda p, m: (0, 0)),
            pl.BlockSpec((H, C), lambda p, m: (0, 0)),
            pl.BlockSpec((1, C), lambda p, m: (0, 0)),
        ],
        out_specs=pl.BlockSpec((BM, C), lambda p, m: (m, 0)),
        out_shape=jax.ShapeDtypeStruct((N, C), jnp.float32),
        scratch_shapes=[
            pltpu.VMEM((N, H), jnp.float32),
            pltpu.VMEM((N, H), jnp.float32),
            pltpu.VMEM((N, C), jnp.float32),
        ],
    )(x, adj, W1, b1.reshape(1, H), W2, b2.reshape(1, C))
    return out
